# Initial kernel scaffold; baseline (speedup 1.0000x reference)
import jax
import jax.numpy as jnp
from jax.experimental import pallas as pl


def kernel(x, edge_index, edge_attr, W1, b1, We1, W2, b2, We2):
    raise NotImplementedError("write your pallas kernel here")


# speedup vs baseline: 8.1223x; 8.1223x over previous
import functools

import jax
import jax.numpy as jnp
from jax import lax
from jax.experimental import pallas as pl
from jax.experimental.pallas import tpu as pltpu
from jax.experimental.pallas import tpu_sc as plsc

N_NODES = 10000
N_PAD = 10240
N_EDGES = 320000
D = 128
DA = 16

NC = 2
NS = 16
NW = NC * NS
EPT = N_EDGES // NW
CH = 80
NCHUNK = EPT // CH
RPS = N_PAD // NS
ZR = 128
BLK = 1024

_MESH = dict(core_axis_name="c", subcore_axis_name="s", num_cores=NC,
             num_subcores=NS)


def _worker_id():
  c = lax.axis_index("c")
  s = lax.axis_index("s")
  return c, s, c * NS + s


def _zero_rows16(ref, nrows):
  z = jnp.zeros((16,), jnp.float32)

  def body(j, carry):
    ref[j, :] = z
    return carry

  lax.fori_loop(0, nrows, body, 0)


def _zero_rows128(ref, nrows):
  z = jnp.zeros((16,), jnp.float32)

  def body(j, carry):
    for k in range(8):
      ref[j, pl.ds(k * 16, 16)] = z
    return carry

  lax.fori_loop(0, nrows, body, 0)


def _sc_degree_body(dst_hbm, deg_out, didx, ones_v, deg_sp):
  c, s, wid = _worker_id()

  _zero_rows16(ones_v, CH)
  for k in range(8):
    pltpu.sync_copy(ones_v, deg_sp.at[pl.ds(s * RPS + k * CH, CH)])

  onehot = jnp.where(lax.iota(jnp.int32, 16) == 0,
                     jnp.float32(1), jnp.float32(0))

  def fill(j, carry):
    ones_v[j, :] = onehot
    return carry

  lax.fori_loop(0, CH, fill, 0)
  plsc.subcore_barrier()

  def chunk(i, carry):
    off = wid * EPT + i * CH
    pltpu.sync_copy(dst_hbm.at[pl.ds(off, CH)], didx)
    pltpu.sync_copy(ones_v, deg_sp.at[didx], add=True)
    return carry

  lax.fori_loop(0, NCHUNK, chunk, 0)
  plsc.subcore_barrier()
  pltpu.sync_copy(deg_sp.at[pl.ds(s * RPS, RPS)],
                  deg_out.at[c, pl.ds(s * RPS, RPS)])


def _sc_degree(dst):
  f = pl.kernel(
      _sc_degree_body,
      out_type=jax.ShapeDtypeStruct((NC, N_PAD, DA), jnp.float32),
      mesh=plsc.VectorSubcoreMesh(**_MESH),
      scratch_types=[
          pltpu.VMEM((CH,), jnp.int32),
          pltpu.VMEM((CH, DA), jnp.float32),
          pltpu.VMEM_SHARED((N_PAD, DA), jnp.float32),
      ],
  )
  return f(dst)


def _sc_agg_body(src_hbm, dst_hbm, ht_hbm, acc_out,
                 sidx, didx, rows, acc_sp, sem):
  c, s, wid = _worker_id()

  _zero_rows128(rows, CH)
  for k in range(8):
    pltpu.sync_copy(rows, acc_sp.at[pl.ds(s * RPS + k * CH, CH)])
  plsc.subcore_barrier()

  def chunk(i, carry):
    off = wid * EPT + i * CH
    pltpu.sync_copy(src_hbm.at[pl.ds(off, CH)], sidx)
    pltpu.sync_copy(dst_hbm.at[pl.ds(off, CH)], didx)
    pltpu.async_copy(ht_hbm.at[sidx], rows, sem).wait()
    pltpu.sync_copy(rows, acc_sp.at[didx], add=True)
    return carry

  lax.fori_loop(0, NCHUNK, chunk, 0)
  plsc.subcore_barrier()
  pltpu.sync_copy(acc_sp.at[pl.ds(s * RPS, RPS)],
                  acc_out.at[c, pl.ds(s * RPS, RPS)])


def _sc_agg(src, dst, ht):
  f = pl.kernel(
      _sc_agg_body,
      out_type=jax.ShapeDtypeStruct((NC, N_PAD, D), jnp.float32),
      mesh=plsc.VectorSubcoreMesh(**_MESH),
      scratch_types=[
          pltpu.VMEM((CH,), jnp.int32),
          pltpu.VMEM((CH,), jnp.int32),
          pltpu.VMEM((CH, D), jnp.float32),
          pltpu.VMEM_SHARED((N_PAD, D), jnp.float32),
          pltpu.SemaphoreType.DMA,
      ],
  )
  return f(src, dst, ht)


def _sc_ea_body(src_hbm, dst_hbm, dinv2_hbm, attr_hbm, eat_out,
                sidx, didx, dv, atb, stg, eat_sp, sem):
  c, s, wid = _worker_id()

  _zero_rows16(stg, CH)
  for k in range(8):
    pltpu.sync_copy(stg, eat_sp.at[pl.ds(s * RPS + k * CH, CH)])
  plsc.subcore_barrier()

  def chunk(i, carry):
    off = wid * EPT + i * CH
    pltpu.sync_copy(src_hbm.at[pl.ds(off, CH)], sidx)
    pltpu.sync_copy(dst_hbm.at[pl.ds(off, CH)], didx)
    pltpu.async_copy(dinv2_hbm.at[sidx], dv, sem).wait()
    pltpu.sync_copy(attr_hbm.at[pl.ds(off, CH)], atb)

    def mul(j, carry2):
      stg[j, :] = dv[j, pl.ds(0, DA)] * atb[j, :]
      return carry2

    lax.fori_loop(0, CH, mul, 0)
    pltpu.sync_copy(stg, eat_sp.at[didx], add=True)
    return carry

  lax.fori_loop(0, NCHUNK, chunk, 0)
  plsc.subcore_barrier()
  pltpu.sync_copy(eat_sp.at[pl.ds(s * RPS, RPS)],
                  eat_out.at[c, pl.ds(s * RPS, RPS)])


def _sc_ea(src, dst, dinv2, attr):
  f = pl.kernel(
      _sc_ea_body,
      out_type=jax.ShapeDtypeStruct((NC, N_PAD, DA), jnp.float32),
      mesh=plsc.VectorSubcoreMesh(**_MESH),
      scratch_types=[
          pltpu.VMEM((CH,), jnp.int32),
          pltpu.VMEM((CH,), jnp.int32),
          pltpu.VMEM((CH, D), jnp.float32),
          pltpu.VMEM((CH, DA), jnp.float32),
          pltpu.VMEM((CH, DA), jnp.float32),
          pltpu.VMEM_SHARED((N_PAD, DA), jnp.float32),
          pltpu.SemaphoreType.DMA,
      ],
  )
  return f(src, dst, dinv2, attr)


def _tc_b_body(x_ref, w1_ref, b1_ref, d0_ref, d1_ref, dinv2_ref, h1t_ref):
  deg = d0_ref[:, :1] + d1_ref[:, :1] + 1.0
  dinv = lax.rsqrt(deg)
  dinv2_ref[...] = jnp.broadcast_to(dinv, (BLK, D))
  h = jnp.dot(x_ref[...], w1_ref[...], precision=lax.Precision.HIGHEST,
              preferred_element_type=jnp.float32) + b1_ref[...]
  h1t_ref[...] = dinv * h


def _tc_b(x, W1, b1r, d0, d1):
  row = lambda i: (i, 0)
  fixed = lambda i: (0, 0)
  return pl.pallas_call(
      _tc_b_body,
      grid=(N_PAD // BLK,),
      in_specs=[
          pl.BlockSpec((BLK, D), row),
          pl.BlockSpec((D, D), fixed),
          pl.BlockSpec((1, D), fixed),
          pl.BlockSpec((BLK, DA), row),
          pl.BlockSpec((BLK, DA), row),
      ],
      out_specs=[
          pl.BlockSpec((BLK, D), row),
          pl.BlockSpec((BLK, D), row),
      ],
      out_shape=[
          jax.ShapeDtypeStruct((N_PAD, D), jnp.float32),
          jax.ShapeDtypeStruct((N_PAD, D), jnp.float32),
      ],
  )(x, W1, b1r, d0, d1)


def _tc_d_body(a0, a1, e0, e1, h1t, dinv2, we1, w2, b2, we2,
               h2t_ref, e2_ref):
  dinvc = dinv2[:, :1]
  ea = dinv2[:, :DA] * (e0[...] + e1[...])
  edge1 = jnp.dot(ea, we1[...], precision=lax.Precision.HIGHEST,
                  preferred_element_type=jnp.float32)
  out1 = jnp.maximum(dinvc * (a0[...] + a1[...] + h1t[...]) + edge1, 0.0)
  h2 = jnp.dot(out1, w2[...], precision=lax.Precision.HIGHEST,
               preferred_element_type=jnp.float32) + b2[...]
  h2t_ref[...] = dinvc * h2
  e2_ref[...] = jnp.dot(ea, we2[...], precision=lax.Precision.HIGHEST,
                        preferred_element_type=jnp.float32)


def _tc_d(a0, a1, e0, e1, h1t, dinv2, We1, W2, b2r, We2):
  row = lambda i: (i, 0)
  fixed = lambda i: (0, 0)
  return pl.pallas_call(
      _tc_d_body,
      grid=(N_PAD // BLK,),
      in_specs=[
          pl.BlockSpec((BLK, D), row),
          pl.BlockSpec((BLK, D), row),
          pl.BlockSpec((BLK, DA), row),
          pl.BlockSpec((BLK, DA), row),
          pl.BlockSpec((BLK, D), row),
          pl.BlockSpec((BLK, D), row),
          pl.BlockSpec((DA, D), fixed),
          pl.BlockSpec((D, D), fixed),
          pl.BlockSpec((1, D), fixed),
          pl.BlockSpec((DA, D), fixed),
      ],
      out_specs=[
          pl.BlockSpec((BLK, D), row),
          pl.BlockSpec((BLK, D), row),
      ],
      out_shape=[
          jax.ShapeDtypeStruct((N_PAD, D), jnp.float32),
          jax.ShapeDtypeStruct((N_PAD, D), jnp.float32),
      ],
  )(a0, a1, e0, e1, h1t, dinv2, We1, W2, b2r, We2)


def _tc_f_body(a0, a1, h2t, e2, dinv2, out_ref):
  out_ref[...] = dinv2[:, :1] * (a0[...] + a1[...] + h2t[...]) + e2[...]


def _tc_f(a0, a1, h2t, e2, dinv2):
  row = lambda i: (i, 0)
  return pl.pallas_call(
      _tc_f_body,
      grid=(N_PAD // BLK,),
      in_specs=[
          pl.BlockSpec((BLK, D), row),
          pl.BlockSpec((BLK, D), row),
          pl.BlockSpec((BLK, D), row),
          pl.BlockSpec((BLK, D), row),
          pl.BlockSpec((BLK, D), row),
      ],
      out_specs=pl.BlockSpec((BLK, D), row),
      out_shape=jax.ShapeDtypeStruct((N_PAD, D), jnp.float32),
  )(a0, a1, h2t, e2, dinv2)


def kernel(x, edge_index, edge_attr, W1, b1, We1, W2, b2, We2):
  src = edge_index[0]
  dst = edge_index[1]
  b1r = b1.reshape(1, D)
  b2r = b2.reshape(1, D)
  xp = jnp.pad(x, ((0, N_PAD - N_NODES), (0, 0)))

  degp = _sc_degree(dst)
  dinv2, h1t = _tc_b(xp, W1, b1r, degp[0], degp[1])
  accp = _sc_agg(src, dst, h1t)
  eatp = _sc_ea(src, dst, dinv2, edge_attr)
  h2t, e2 = _tc_d(accp[0], accp[1], eatp[0], eatp[1], h1t, dinv2,
                  We1, W2, b2r, We2)
  acc2p = _sc_agg(src, dst, h2t)
  return _tc_f(acc2p[0], acc2p[1], h2t, e2, dinv2)[:N_NODES]
